# baseline (speedup 1.0000x reference)
import functools

import jax
import jax.numpy as jnp
from jax import lax
from jax.experimental import pallas as pl
from jax.experimental.pallas import tpu as pltpu
from jax.experimental.pallas import tpu_sc as plsc

B = 8
L = 256
C = 8
D = 256
H = 256
NB = 2 * B
N_NODES = L * NB
N_ROWS = 9 * N_NODES
N_WORKERS = 32
ROWS_PER_W = N_ROWS // N_WORKERS
GCHUNK = 128
N_CHUNKS = ROWS_PER_W // GCHUNK
_BLKN = 128
_NBLK = N_NODES // _BLKN

_BF = jnp.bfloat16
_F = jnp.float32


def _split_w(w):
    return w, jnp.zeros((1, 1), _F)


def _dot3(x, w_hi, w_lo):
    return jnp.dot(x, w_hi, preferred_element_type=_F,
                   precision=jax.lax.Precision.HIGHEST)


def _xla_tanh(x):
    xc = jnp.clip(x, -7.90531110763549805, 7.90531110763549805)
    x2 = xc * xc
    p = x2 * (-2.76076847742355e-16) + 2.00018790482477e-13
    p = x2 * p + (-8.60467152213735e-11)
    p = x2 * p + 5.12229709037114e-08
    p = x2 * p + 1.48572235717979e-05
    p = x2 * p + 6.37261928875436e-04
    p = x2 * p + 4.89352455891786e-03
    num = xc * p
    q = x2 * 1.19825839466702e-06 + 1.18534705686654e-04
    q = x2 * q + 2.26843463243900e-03
    q = x2 * q + 4.89352518554385e-03
    res = num / q
    return jnp.where(jnp.abs(x) < 0.0004, x, res)


@functools.lru_cache(maxsize=1)
def _sc_gather_fn():
    mesh = plsc.VectorSubcoreMesh(core_axis_name="c", subcore_axis_name="s",
                                  num_cores=2)

    @functools.partial(
        pl.kernel,
        out_type=jax.ShapeDtypeStruct((N_ROWS, D), jnp.float32),
        mesh=mesh,
        scratch_types=[
            pltpu.VMEM((GCHUNK,), jnp.int32),
            pltpu.VMEM((GCHUNK, D), jnp.float32),
            pltpu.SemaphoreType.DMA,
        ],
    )
    def gather(idx_hbm, table_hbm, out_hbm, idx_v, rows_v, sem):
        wid = lax.axis_index("s") * 2 + lax.axis_index("c")
        base = wid * ROWS_PER_W

        def chunk(i, carry):
            off = base + i * GCHUNK
            pltpu.sync_copy(idx_hbm.at[pl.ds(off, GCHUNK)], idx_v)
            pltpu.async_copy(table_hbm.at[idx_v], rows_v, sem).wait()
            pltpu.sync_copy(rows_v, out_hbm.at[pl.ds(off, GCHUNK)])
            return carry

        lax.fori_loop(0, N_CHUNKS, chunk, 0)

    return gather


def _sc_gather(idx, table):
    return _sc_gather_fn()(idx, table)


def _fused_body(rows_ref, wnh_ref, wnl_ref, bn_ref,
                wifh0, wifl0, whfh0, whfl0, bf0, bhf0,
                wibh0, wibl0, whbh0, whbl0, bb0, bhb0,
                wifh1, wifl1, whfh1, whfl1, bf1, bhf1,
                wibh1, wibl1, whbh1, whbl1, bb1, bhb1,
                wch_ref, wcl_ref, bc_ref, w2_ref, b2_ref,
                out_ref, seq_buf, gif_ref, gib_ref, h_buf):
    i = pl.program_id(0)

    @pl.when(i < _NBLK)
    def node_phase():
        x = rows_ref[...]
        y = _dot3(x.reshape(9 * _BLKN, D), wnh_ref[...], wnl_ref[...]) + bn_ref[...]
        y = y.reshape(9, _BLKN, D)
        er = y[0]
        maxc = jnp.max(y[1:], axis=0)
        sumc = jnp.sum(y[1:], axis=0)
        blk = pl.multiple_of(i * _BLKN, _BLKN)
        seq_buf[pl.ds(blk, _BLKN), :] = jnp.maximum(
            jnp.maximum(0.0, maxc), er + sumc)

    @pl.when(i == _NBLK)
    def gru_phase():
        def sigm(x):
            return 0.5 + 0.5 * jnp.tanh(0.5 * x)

        def cell(h, gi, gh):
            r = sigm(gi[:, 0:H] + gh[:, 0:H])
            z = sigm(gi[:, H:2 * H] + gh[:, H:2 * H])
            n = jnp.tanh(gi[:, 2 * H:3 * H] + r * gh[:, 2 * H:3 * H])
            return (1.0 - z) * n + z * h

        def dot3_chunked(out_ref, x_ref, ncols, wh, wl, bias):
            ch = 512

            def body(k, carry):
                rr = pl.multiple_of(k * ch, ch)
                xc = x_ref[pl.ds(rr, ch), 0:ncols]
                out_ref[pl.ds(rr, ch), :] = _dot3(xc, wh, wl) + bias
                return carry

            lax.fori_loop(0, N_NODES // ch, body, 0)

        def bigru(x_ref, ncols, wifh, wifl, bf, wibh, wibl, bb,
                  whfh, whfl, bhf_r, whbh, whbl, bhb_r):
            dot3_chunked(gif_ref, x_ref, ncols, wifh[...], wifl[...], bf[...])
            dot3_chunked(gib_ref, x_ref, ncols, wibh[...], wibl[...], bb[...])
            whf_hi = whfh[...]
            whf_lo = whfl[...]
            whb_hi = whbh[...]
            whb_lo = whbl[...]
            bhf = bhf_r[...]
            bhb = bhb_r[...]

            def step(t, carry):
                h_f, h_b = carry
                tf = pl.multiple_of(t * NB, NB)
                tb = pl.multiple_of((L - 1 - t) * NB, NB)
                gf = gif_ref[pl.ds(tf, NB), :]
                gb = gib_ref[pl.ds(tb, NB), :]
                gh_f = _dot3(h_f, whf_hi, whf_lo) + bhf
                gh_b = _dot3(h_b, whb_hi, whb_lo) + bhb
                h_f = cell(h_f, gf, gh_f)
                h_b = cell(h_b, gb, gh_b)
                h_buf[pl.ds(tf, NB), 0:H] = h_f
                h_buf[pl.ds(tb, NB), H:2 * H] = h_b
                return (h_f, h_b)

            h0 = jnp.zeros((NB, H), jnp.float32)
            lax.fori_loop(0, L, step, (h0, h0))

        bigru(seq_buf, D, wifh0, wifl0, bf0, wibh0, wibl0, bb0,
              whfh0, whfl0, bhf0, whbh0, whbl0, bhb0)
        bigru(h_buf, 2 * H, wifh1, wifl1, bf1, wibh1, wibl1, bb1,
              whfh1, whfl1, bhf1, whbh1, whbl1, bhb1)

        def comb_body(k, carry):
            rr = pl.multiple_of(k * 512, 512)
            xc = h_buf[pl.ds(rr, 512), :]
            seq_buf[pl.ds(rr, 512), :] = _dot3(xc, wch_ref[...],
                                               wcl_ref[...]) + bc_ref[...]
            return carry

        lax.fori_loop(0, N_NODES // 512, comb_body, 0)

        def red(t, carry):
            m, s = carry
            tt = pl.multiple_of(t * NB, NB)
            blk = seq_buf[pl.ds(tt, NB), :]
            return jnp.maximum(m, blk), s + blk

        init = seq_buf[pl.ds(0, NB), :]
        m, s = lax.fori_loop(1, L, red, (init, init))
        z1 = jnp.sum(m[0:B] * m[B:2 * B], axis=1, keepdims=True)
        z2 = jnp.sum(s[0:B] * s[B:2 * B] * w2_ref[...], axis=1,
                     keepdims=True) + b2_ref[0, 0]
        out_ref[...] = z1 + z2


def _fused_call(rows3, args):
    full = lambda shape: pl.BlockSpec(shape, lambda i: tuple(0 for _ in shape))
    in_specs = [pl.BlockSpec((9, _BLKN, D),
                             lambda i: (0, jnp.minimum(i, _NBLK - 1), 0))]
    in_specs += [full(a.shape) for a in args]
    return pl.pallas_call(
        _fused_body,
        grid=(_NBLK + 1,),
        in_specs=in_specs,
        out_specs=pl.BlockSpec((B, 1), lambda i: (0, 0)),
        out_shape=jax.ShapeDtypeStruct((B, 1), jnp.float32),
        scratch_shapes=[
            pltpu.VMEM((N_NODES, D), jnp.float32),
            pltpu.VMEM((N_NODES, 3 * H), jnp.float32),
            pltpu.VMEM((N_NODES, 3 * H), jnp.float32),
            pltpu.VMEM((N_NODES, 2 * H), jnp.float32),
        ],
    )(rows3, *args)


def kernel(root1, child1, root2, child2, embed, W_lin, b_lin,
           Wih_l0_f, Whh_l0_f, bih_l0_f, bhh_l0_f,
           Wih_l0_b, Whh_l0_b, bih_l0_b, bhh_l0_b,
           Wih_l1_f, Whh_l1_f, bih_l1_f, bhh_l1_f,
           Wih_l1_b, Whh_l1_b, bih_l1_b, bhh_l1_b,
           W_comb, b_comb, W2, b2):
    root = jnp.stack([root1, root2])
    child = jnp.stack([child1, child2])
    root_t = root.transpose(2, 0, 1).reshape(1, L, NB)
    child_t = child.transpose(3, 2, 0, 1).reshape(C, L, NB)
    idx = jnp.concatenate([root_t, child_t], axis=0).reshape(-1)
    idx = idx.astype(jnp.int32)

    rows = _sc_gather(idx, embed)
    rows3 = rows.reshape(9, N_NODES, D)

    wnh, wnl = _split_w(W_lin.T)
    wch, wcl = _split_w(W_comb.T)

    def layer(Wif, Whf, bif, bhf, Wib, Whb, bib, bhb):
        wifh, wifl = _split_w(Wif.T)
        whfh, whfl = _split_w(Whf.T)
        wibh, wibl = _split_w(Wib.T)
        whbh, whbl = _split_w(Whb.T)
        return (wifh, wifl, whfh, whfl, bif.reshape(1, -1), bhf.reshape(1, -1),
                wibh, wibl, whbh, whbl, bib.reshape(1, -1), bhb.reshape(1, -1))

    args = ((wnh, wnl, b_lin.reshape(1, D))
            + layer(Wih_l0_f, Whh_l0_f, bih_l0_f, bhh_l0_f,
                    Wih_l0_b, Whh_l0_b, bih_l0_b, bhh_l0_b)
            + layer(Wih_l1_f, Whh_l1_f, bih_l1_f, bhh_l1_f,
                    Wih_l1_b, Whh_l1_b, bih_l1_b, bhh_l1_b)
            + (wch, wcl, b_comb.reshape(1, H), W2, b2.reshape(1, 1)))

    out = _fused_call(rows3, args)
    return out.reshape(B)


# speedup vs baseline: 2.6368x; 2.6368x over previous
import functools

import jax
import jax.numpy as jnp
from jax import lax
from jax.experimental import pallas as pl
from jax.experimental.pallas import tpu as pltpu
from jax.experimental.pallas import tpu_sc as plsc

B = 8
L = 256
C = 8
D = 256
H = 256
NB = 2 * B
N_NODES = L * NB
N_ROWS = 9 * N_NODES
N_WORKERS = 32
ROWS_PER_W = N_ROWS // N_WORKERS
GCHUNK = 128
N_CHUNKS = ROWS_PER_W // GCHUNK
_BLKN = 128
_NBLK = N_NODES // _BLKN

_BF = jnp.bfloat16
_F = jnp.float32


def _split_w(w):
    return w, jnp.zeros((1, 1), _F)


def _dot3(x, w_hi, w_lo):
    return jnp.dot(x, w_hi, preferred_element_type=_F)


def _xla_tanh(x):
    xc = jnp.clip(x, -7.90531110763549805, 7.90531110763549805)
    x2 = xc * xc
    p = x2 * (-2.76076847742355e-16) + 2.00018790482477e-13
    p = x2 * p + (-8.60467152213735e-11)
    p = x2 * p + 5.12229709037114e-08
    p = x2 * p + 1.48572235717979e-05
    p = x2 * p + 6.37261928875436e-04
    p = x2 * p + 4.89352455891786e-03
    num = xc * p
    q = x2 * 1.19825839466702e-06 + 1.18534705686654e-04
    q = x2 * q + 2.26843463243900e-03
    q = x2 * q + 4.89352518554385e-03
    res = num / q
    return jnp.where(jnp.abs(x) < 0.0004, x, res)


@functools.lru_cache(maxsize=1)
def _sc_gather_fn():
    mesh = plsc.VectorSubcoreMesh(core_axis_name="c", subcore_axis_name="s",
                                  num_cores=2)

    @functools.partial(
        pl.kernel,
        out_type=jax.ShapeDtypeStruct((N_ROWS, D), jnp.float32),
        mesh=mesh,
        scratch_types=[
            pltpu.VMEM((GCHUNK,), jnp.int32),
            pltpu.VMEM((GCHUNK, D), jnp.float32),
            pltpu.SemaphoreType.DMA,
        ],
    )
    def gather(idx_hbm, table_hbm, out_hbm, idx_v, rows_v, sem):
        wid = lax.axis_index("s") * 2 + lax.axis_index("c")
        base = wid * ROWS_PER_W

        def chunk(i, carry):
            off = base + i * GCHUNK
            pltpu.sync_copy(idx_hbm.at[pl.ds(off, GCHUNK)], idx_v)
            pltpu.async_copy(table_hbm.at[idx_v], rows_v, sem).wait()
            pltpu.sync_copy(rows_v, out_hbm.at[pl.ds(off, GCHUNK)])
            return carry

        lax.fori_loop(0, N_CHUNKS, chunk, 0)

    return gather


def _sc_gather(idx, table):
    return _sc_gather_fn()(idx, table)


def _fused_body(rows_ref, wnh_ref, wnl_ref, bn_ref,
                wifh0, wifl0, whfh0, whfl0, bf0, bhf0,
                wibh0, wibl0, whbh0, whbl0, bb0, bhb0,
                wifh1, wifl1, whfh1, whfl1, bf1, bhf1,
                wibh1, wibl1, whbh1, whbl1, bb1, bhb1,
                wch_ref, wcl_ref, bc_ref, w2_ref, b2_ref,
                out_ref, seq_buf, gif_ref, gib_ref, h_buf):
    i = pl.program_id(0)

    @pl.when(i < _NBLK)
    def node_phase():
        x = rows_ref[...]
        y = _dot3(x.reshape(9 * _BLKN, D), wnh_ref[...], wnl_ref[...]) + bn_ref[...]
        y = y.reshape(9, _BLKN, D)
        er = y[0]
        maxc = jnp.max(y[1:], axis=0)
        sumc = jnp.sum(y[1:], axis=0)
        blk = pl.multiple_of(i * _BLKN, _BLKN)
        seq_buf[pl.ds(blk, _BLKN), :] = jnp.maximum(
            jnp.maximum(0.0, maxc), er + sumc)

    @pl.when(i == _NBLK)
    def gru_phase():
        def sigm(x):
            return 0.5 + 0.5 * jnp.tanh(0.5 * x)

        def cell(h, gi, gh):
            r = sigm(gi[:, 0:H] + gh[:, 0:H])
            z = sigm(gi[:, H:2 * H] + gh[:, H:2 * H])
            n = jnp.tanh(gi[:, 2 * H:3 * H] + r * gh[:, 2 * H:3 * H])
            return (1.0 - z) * n + z * h

        def dot3_chunked(out_ref, x_ref, ncols, wh, wl, bias):
            ch = 512

            def body(k, carry):
                rr = pl.multiple_of(k * ch, ch)
                xc = x_ref[pl.ds(rr, ch), 0:ncols]
                out_ref[pl.ds(rr, ch), :] = _dot3(xc, wh, wl) + bias
                return carry

            lax.fori_loop(0, N_NODES // ch, body, 0)

        def bigru(x_ref, ncols, wifh, wifl, bf, wibh, wibl, bb,
                  whfh, whfl, bhf_r, whbh, whbl, bhb_r):
            dot3_chunked(gif_ref, x_ref, ncols, wifh[...], wifl[...], bf[...])
            dot3_chunked(gib_ref, x_ref, ncols, wibh[...], wibl[...], bb[...])
            whf_hi = whfh[...]
            whf_lo = whfl[...]
            whb_hi = whbh[...]
            whb_lo = whbl[...]
            bhf = bhf_r[...]
            bhb = bhb_r[...]

            def step(t, carry):
                h_f, h_b = carry
                tf = pl.multiple_of(t * NB, NB)
                tb = pl.multiple_of((L - 1 - t) * NB, NB)
                gf = gif_ref[pl.ds(tf, NB), :]
                gb = gib_ref[pl.ds(tb, NB), :]
                gh_f = _dot3(h_f, whf_hi, whf_lo) + bhf
                gh_b = _dot3(h_b, whb_hi, whb_lo) + bhb
                h_f = cell(h_f, gf, gh_f)
                h_b = cell(h_b, gb, gh_b)
                h_buf[pl.ds(tf, NB), 0:H] = h_f
                h_buf[pl.ds(tb, NB), H:2 * H] = h_b
                return (h_f, h_b)

            h0 = jnp.zeros((NB, H), jnp.float32)
            lax.fori_loop(0, L, step, (h0, h0))

        bigru(seq_buf, D, wifh0, wifl0, bf0, wibh0, wibl0, bb0,
              whfh0, whfl0, bhf0, whbh0, whbl0, bhb0)
        bigru(h_buf, 2 * H, wifh1, wifl1, bf1, wibh1, wibl1, bb1,
              whfh1, whfl1, bhf1, whbh1, whbl1, bhb1)

        def comb_body(k, carry):
            rr = pl.multiple_of(k * 512, 512)
            xc = h_buf[pl.ds(rr, 512), :]
            seq_buf[pl.ds(rr, 512), :] = _dot3(xc, wch_ref[...],
                                               wcl_ref[...]) + bc_ref[...]
            return carry

        lax.fori_loop(0, N_NODES // 512, comb_body, 0)

        def red(t, carry):
            m, s = carry
            tt = pl.multiple_of(t * NB, NB)
            blk = seq_buf[pl.ds(tt, NB), :]
            return jnp.maximum(m, blk), s + blk

        init = seq_buf[pl.ds(0, NB), :]
        m, s = lax.fori_loop(1, L, red, (init, init))
        z1 = jnp.sum(m[0:B] * m[B:2 * B], axis=1, keepdims=True)
        z2 = jnp.sum(s[0:B] * s[B:2 * B] * w2_ref[...], axis=1,
                     keepdims=True) + b2_ref[0, 0]
        out_ref[...] = z1 + z2


def _fused_call(rows3, args):
    full = lambda shape: pl.BlockSpec(shape, lambda i: tuple(0 for _ in shape))
    in_specs = [pl.BlockSpec((9, _BLKN, D),
                             lambda i: (0, jnp.minimum(i, _NBLK - 1), 0))]
    in_specs += [full(a.shape) for a in args]
    return pl.pallas_call(
        _fused_body,
        grid=(_NBLK + 1,),
        in_specs=in_specs,
        out_specs=pl.BlockSpec((B, 1), lambda i: (0, 0)),
        out_shape=jax.ShapeDtypeStruct((B, 1), jnp.float32),
        scratch_shapes=[
            pltpu.VMEM((N_NODES, D), jnp.float32),
            pltpu.VMEM((N_NODES, 3 * H), jnp.float32),
            pltpu.VMEM((N_NODES, 3 * H), jnp.float32),
            pltpu.VMEM((N_NODES, 2 * H), jnp.float32),
        ],
    )(rows3, *args)


def kernel(root1, child1, root2, child2, embed, W_lin, b_lin,
           Wih_l0_f, Whh_l0_f, bih_l0_f, bhh_l0_f,
           Wih_l0_b, Whh_l0_b, bih_l0_b, bhh_l0_b,
           Wih_l1_f, Whh_l1_f, bih_l1_f, bhh_l1_f,
           Wih_l1_b, Whh_l1_b, bih_l1_b, bhh_l1_b,
           W_comb, b_comb, W2, b2):
    root = jnp.stack([root1, root2])
    child = jnp.stack([child1, child2])
    root_t = root.transpose(2, 0, 1).reshape(1, L, NB)
    child_t = child.transpose(3, 2, 0, 1).reshape(C, L, NB)
    idx = jnp.concatenate([root_t, child_t], axis=0).reshape(-1)
    idx = idx.astype(jnp.int32)

    rows = _sc_gather(idx, embed)
    rows3 = rows.reshape(9, N_NODES, D)

    wnh, wnl = _split_w(W_lin.T)
    wch, wcl = _split_w(W_comb.T)

    def layer(Wif, Whf, bif, bhf, Wib, Whb, bib, bhb):
        wifh, wifl = _split_w(Wif.T)
        whfh, whfl = _split_w(Whf.T)
        wibh, wibl = _split_w(Wib.T)
        whbh, whbl = _split_w(Whb.T)
        return (wifh, wifl, whfh, whfl, bif.reshape(1, -1), bhf.reshape(1, -1),
                wibh, wibl, whbh, whbl, bib.reshape(1, -1), bhb.reshape(1, -1))

    args = ((wnh, wnl, b_lin.reshape(1, D))
            + layer(Wih_l0_f, Whh_l0_f, bih_l0_f, bhh_l0_f,
                    Wih_l0_b, Whh_l0_b, bih_l0_b, bhh_l0_b)
            + layer(Wih_l1_f, Whh_l1_f, bih_l1_f, bhh_l1_f,
                    Wih_l1_b, Whh_l1_b, bih_l1_b, bhh_l1_b)
            + (wch, wcl, b_comb.reshape(1, H), W2, b2.reshape(1, 1)))

    out = _fused_call(rows3, args)
    return out.reshape(B)
